# baseline (speedup 1.0000x reference)
import jax
import jax.numpy as jnp
from jax import lax
from jax.experimental import pallas as pl
from jax.experimental.pallas import tpu as pltpu
from jax.experimental.pallas import tpu_sc as plsc

_N = 32768
_NV = _N // 16
_NC = 2
_NW = 32
_ROWS_PER_W = 4
_U = 4
_US = 4


def _sc_body(x_hbm, k_hbm, val_hbm, idx_hbm,
             xrow, ubuf, hist1w, hist2, colsum2, hist3, colsum3,
             kbuf, resv, resi):
    imin = jnp.int32(-2147483648)
    lane = lax.iota(jnp.int32, 16)
    c31 = jnp.full((16,), 31, jnp.int32)
    cimin = jnp.full((16,), imin, jnp.int32)
    ones = jnp.ones((16,), jnp.int32)
    zeros16 = jnp.zeros((16,), jnp.int32)
    laneoff = lane * 2065

    wid = lax.axis_index("s") * _NC + lax.axis_index("c")

    pltpu.sync_copy(k_hbm, kbuf)
    kv = kbuf[...]

    def zinit(i, c):
        hist1w[pl.ds(i * 16, 16)] = zeros16
        return c
    lax.fori_loop(0, 2065, zinit, 0)

    def zinit2(i, c):
        hist2[pl.ds(i * 16, 16)] = zeros16
        colsum2[pl.ds(i * 16, 16)] = zeros16
        return c
    lax.fori_loop(0, 128, zinit2, 0)

    def scan_hist(href, csref, nbins, kcur, zero_after):
        def body(i, carry):
            tot_carry, b_acc, cb_acc, cnt_acc, col_acc = carry
            for j in range(_US):
                ii = i * _US + j
                hv = href[pl.ds(ii * 16, 16)]
                if csref is not None:
                    cv = csref[pl.ds(ii * 16, 16)]
                if zero_after:
                    href[pl.ds(ii * 16, 16)] = zeros16
                    if csref is not None:
                        csref[pl.ds(ii * 16, 16)] = zeros16
                cs = plsc.cumsum(hv)
                tot = cs + tot_carry
                excl = tot - hv
                sel = (tot >= kcur) & (excl < kcur)
                gidx = lane + ii * 16
                b_acc = b_acc + jnp.where(sel, gidx, 0)
                cb_acc = cb_acc + jnp.where(sel, excl, 0)
                cnt_acc = cnt_acc + jnp.where(sel, hv, 0)
                if csref is not None:
                    col_acc = col_acc + jnp.where(sel, cv, 0)
                tot_carry = tot_carry + jnp.max(cs)
            return tot_carry, b_acc, cb_acc, cnt_acc, col_acc
        _, b_acc, cb_acc, cnt_acc, col_acc = lax.fori_loop(
            0, nbins // (16 * _US), body, (zeros16,) * 5)
        return (jnp.max(b_acc), jnp.max(cb_acc), jnp.max(cnt_acc),
                jnp.max(col_acc))

    def scan_hist_wide(kcur):
        def body(i, carry):
            tot_carry, b_acc, cb_acc, cnt_acc = carry
            hv = zeros16
            for L in range(16):
                off = L * 2065 + i * 16
                hv = hv + hist1w[pl.ds(off, 16)]
                hist1w[pl.ds(off, 16)] = zeros16
            cs = plsc.cumsum(hv)
            tot = cs + tot_carry
            excl = tot - hv
            sel = (tot >= kcur) & (excl < kcur)
            gidx = lane + i * 16
            b_acc = b_acc + jnp.where(sel, gidx, 0)
            cb_acc = cb_acc + jnp.where(sel, excl, 0)
            cnt_acc = cnt_acc + jnp.where(sel, hv, 0)
            return tot_carry + jnp.max(cs), b_acc, cb_acc, cnt_acc
        _, b_acc, cb_acc, cnt_acc = lax.fori_loop(
            0, 128, body, (zeros16,) * 4)
        return jnp.max(b_acc), jnp.max(cb_acc), jnp.max(cnt_acc)

    def row_body(r, accs):
        val_acc, idx_acc = accs
        with jax.named_scope("dma_row"):
            pltpu.sync_copy(x_hbm.at[wid * _ROWS_PER_W + r], xrow)

        def p1(i, c):
            for j in range(_U):
                ii = i * _U + j
                xv = xrow[pl.ds(ii * 16, 16)]
                b = lax.bitcast_convert_type(xv, jnp.int32)
                asr = lax.shift_right_arithmetic(b, c31)
                u = lax.bitwise_xor(b, lax.bitwise_or(asr, cimin))
                ubuf[pl.ds(ii * 16, 16)] = u
                plsc.addupdate_scatter(
                    hist1w, [lax.shift_right_logical(u, 21) + laneoff], ones)
            return c
        with jax.named_scope("p1"):
            lax.fori_loop(0, _NV // _U, p1, 0)
        with jax.named_scope("scan1"):
            b1, cb1, cnt1 = scan_hist_wide(kv)
        k2 = kv - cb1

        def p2(i, c):
            for j in range(_U):
                ii = i * _U + j
                u = ubuf[pl.ds(ii * 16, 16)]
                d1 = lax.shift_right_logical(u, 21)
                d2 = lax.bitwise_and(lax.shift_right_logical(u, 10), 0x7FF)
                m = d1 == b1
                colv = lane + ii * 16
                plsc.addupdate_scatter(hist2, [d2], ones, mask=m)
                plsc.addupdate_scatter(colsum2, [d2], colv, mask=m)
            return c
        with jax.named_scope("p2"):
            lax.fori_loop(0, _NV // _U, p2, 0)
        with jax.named_scope("scan2"):
            b2, cb2, cnt2, col2 = scan_hist(hist2, colsum2, 2048, k2, True)
        k3 = k2 - cb2

        def fast_case(_):
            uv = plsc.load_gather(ubuf, [jnp.broadcast_to(col2, (16,))])
            return uv, col2

        def slow_case(_):
            def z3(i, c):
                hist3[pl.ds(i * 16, 16)] = zeros16
                colsum3[pl.ds(i * 16, 16)] = zeros16
                return c
            lax.fori_loop(0, 64, z3, 0)

            def p3(i, c):
                for j in range(_U):
                    ii = i * _U + j
                    u = ubuf[pl.ds(ii * 16, 16)]
                    d1 = lax.shift_right_logical(u, 21)
                    d2 = lax.bitwise_and(
                        lax.shift_right_logical(u, 10), 0x7FF)
                    d3 = lax.bitwise_and(u, 0x3FF)
                    m = (d1 == b1) & (d2 == b2)
                    colv = lane + ii * 16
                    plsc.addupdate_scatter(hist3, [d3], ones, mask=m)
                    plsc.addupdate_scatter(colsum3, [d3], colv, mask=m)
                return c
            lax.fori_loop(0, _NV // _U, p3, 0)
            b3, cb3, cnt3, col3 = scan_hist(hist3, colsum3, 1024, k3, False)
            ustar = lax.bitwise_or(
                lax.bitwise_or(lax.shift_left(b1, 21), lax.shift_left(b2, 10)),
                b3)
            ustar_v = jnp.broadcast_to(ustar, (16,))

            def tie_case(_):
                m0 = k3 - cb3 - 1

                def lbody(i, carry):
                    eqcnt, ans = carry
                    u = ubuf[pl.ds(i * 16, 16)]
                    meq = u == ustar_v
                    csv = plsc.cumsum(meq.astype(jnp.int32))
                    sel = meq & ((csv + eqcnt) == (m0 + 1))
                    colv = lane + i * 16
                    ans = jnp.maximum(ans, jnp.where(sel, colv, -1))
                    eqcnt = eqcnt + plsc.all_reduce_population_count(meq)
                    return eqcnt, ans
                _, ans = lax.fori_loop(
                    0, _NV, lbody,
                    (zeros16, jnp.full((16,), -1, jnp.int32)))
                return jnp.max(ans)

            col = lax.cond(cnt3 == 1, lambda _: col3, tie_case, 0)
            return ustar_v, col

        with jax.named_scope("resolve"):
            uv, col = lax.cond(cnt2 == 1, fast_case, slow_case, 0)
        bits_v = jnp.where(uv < 0, lax.bitwise_xor(uv, cimin),
                           lax.bitwise_not(uv))
        val_v = lax.bitcast_convert_type(bits_v, jnp.float32)
        val_acc = jnp.where(lane == r, val_v, val_acc)
        idx_acc = jnp.where(lane == r, col, idx_acc)
        return val_acc, idx_acc

    val_acc, idx_acc = lax.fori_loop(
        0, _ROWS_PER_W, row_body,
        (jnp.zeros((16,), jnp.float32), jnp.zeros((16,), jnp.int32)))

    resv[...] = val_acc
    resi[...] = idx_acc
    pltpu.sync_copy(resv, val_hbm.at[wid])
    pltpu.sync_copy(resi, idx_hbm.at[wid])


def _kth_select_sc(x, k_arr):
    mesh = plsc.VectorSubcoreMesh(core_axis_name="c", subcore_axis_name="s")
    f = pl.kernel(
        _sc_body,
        out_type=[
            jax.ShapeDtypeStruct((_NW, 16), jnp.float32),
            jax.ShapeDtypeStruct((_NW, 16), jnp.int32),
        ],
        mesh=mesh,
        compiler_params=pltpu.CompilerParams(needs_layout_passes=False),
        scratch_types=[
            pltpu.VMEM((_N,), jnp.float32),
            pltpu.VMEM((_N,), jnp.int32),
            pltpu.VMEM((33040,), jnp.int32),
            pltpu.VMEM((2048,), jnp.int32),
            pltpu.VMEM((2048,), jnp.int32),
            pltpu.VMEM((1024,), jnp.int32),
            pltpu.VMEM((1024,), jnp.int32),
            pltpu.VMEM((16,), jnp.int32),
            pltpu.VMEM((16,), jnp.float32),
            pltpu.VMEM((16,), jnp.int32),
        ],
    )
    return f(x, k_arr)


def kernel(x, k, dim, keepdim, values, indices):
    k_arr = jnp.full((16,), jnp.asarray(k, jnp.int32))
    vals, idxs = _kth_select_sc(x, k_arr)
    kth_val = vals[:, :_ROWS_PER_W].reshape(128, 1)
    kth_idx = idxs[:, :_ROWS_PER_W].reshape(128, 1)
    zero = (jnp.asarray(dim, jnp.int32) - 1) + (
        jnp.asarray(keepdim).astype(jnp.int32) - 1)
    kth_val = (kth_val + zero.astype(kth_val.dtype)).astype(values.dtype)
    kth_idx = (kth_idx + zero).astype(indices.dtype)
    return kth_val, kth_idx


# speedup vs baseline: 1.3981x; 1.3981x over previous
import jax
import jax.numpy as jnp
from jax import lax
from jax.experimental import pallas as pl
from jax.experimental.pallas import tpu as pltpu
from jax.experimental.pallas import tpu_sc as plsc

_N = 32768
_NV = _N // 16
_NC = 2
_NW = 32
_ROWS_PER_W = 3
_SC_ROWS = _NW * _ROWS_PER_W
_TC_ROWS = 128 - _SC_ROWS
_TC_BLOCK = 8
_U = 4
_US = 4


def _sc_body(x_hbm, k_hbm, val_hbm, idx_hbm,
             xrow, ubuf, hist1, hist2, colsum2, hist3, colsum3,
             kbuf, resv, resi):
    imin = jnp.int32(-2147483648)
    lane = lax.iota(jnp.int32, 16)
    c31 = jnp.full((16,), 31, jnp.int32)
    cimin = jnp.full((16,), imin, jnp.int32)
    ones = jnp.ones((16,), jnp.int32)
    zeros16 = jnp.zeros((16,), jnp.int32)

    wid = lax.axis_index("s") * _NC + lax.axis_index("c")

    pltpu.sync_copy(k_hbm, kbuf)
    kv = kbuf[...]

    def zinit(i, c):
        hist1[pl.ds(i * 16, 16)] = zeros16
        hist2[pl.ds(i * 16, 16)] = zeros16
        colsum2[pl.ds(i * 16, 16)] = zeros16
        return c
    lax.fori_loop(0, 128, zinit, 0)

    def scan_hist(href, csref, nbins, kcur, zero_after):
        def body(i, carry):
            tot_carry, b_acc, cb_acc, cnt_acc, col_acc = carry
            for j in range(_US):
                ii = i * _US + j
                hv = href[pl.ds(ii * 16, 16)]
                if csref is not None:
                    cv = csref[pl.ds(ii * 16, 16)]
                if zero_after:
                    href[pl.ds(ii * 16, 16)] = zeros16
                    if csref is not None:
                        csref[pl.ds(ii * 16, 16)] = zeros16
                cs = plsc.cumsum(hv)
                tot = cs + tot_carry
                excl = tot - hv
                sel = (tot >= kcur) & (excl < kcur)
                gidx = lane + ii * 16
                b_acc = b_acc + jnp.where(sel, gidx, 0)
                cb_acc = cb_acc + jnp.where(sel, excl, 0)
                cnt_acc = cnt_acc + jnp.where(sel, hv, 0)
                if csref is not None:
                    col_acc = col_acc + jnp.where(sel, cv, 0)
                tot_carry = tot_carry + jnp.max(cs)
            return tot_carry, b_acc, cb_acc, cnt_acc, col_acc
        _, b_acc, cb_acc, cnt_acc, col_acc = lax.fori_loop(
            0, nbins // (16 * _US), body, (zeros16,) * 5)
        return (jnp.max(b_acc), jnp.max(cb_acc), jnp.max(cnt_acc),
                jnp.max(col_acc))

    def row_body(r, accs):
        val_acc, idx_acc = accs
        with jax.named_scope("dma_row"):
            pltpu.sync_copy(x_hbm.at[wid * _ROWS_PER_W + r], xrow)

        def p1(i, c):
            for j in range(_U):
                ii = i * _U + j
                xv = xrow[pl.ds(ii * 16, 16)]
                b = lax.bitcast_convert_type(xv, jnp.int32)
                asr = lax.shift_right_arithmetic(b, c31)
                u = lax.bitwise_xor(b, lax.bitwise_or(asr, cimin))
                ubuf[pl.ds(ii * 16, 16)] = u
                plsc.addupdate_scatter(
                    hist1, [lax.shift_right_logical(u, 21)], ones)
            return c
        with jax.named_scope("p1"):
            lax.fori_loop(0, _NV // _U, p1, 0)
        with jax.named_scope("scan1"):
            b1, cb1, cnt1, _ = scan_hist(hist1, None, 2048, kv, True)
        k2 = kv - cb1

        def p2(i, c):
            for j in range(_U):
                ii = i * _U + j
                u = ubuf[pl.ds(ii * 16, 16)]
                d1 = lax.shift_right_logical(u, 21)
                d2 = lax.bitwise_and(lax.shift_right_logical(u, 10), 0x7FF)
                m = d1 == b1
                colv = lane + ii * 16
                plsc.addupdate_scatter(hist2, [d2], ones, mask=m)
                plsc.addupdate_scatter(colsum2, [d2], colv, mask=m)
            return c
        with jax.named_scope("p2"):
            lax.fori_loop(0, _NV // _U, p2, 0)
        with jax.named_scope("scan2"):
            b2, cb2, cnt2, col2 = scan_hist(hist2, colsum2, 2048, k2, True)
        k3 = k2 - cb2

        def fast_case(_):
            uv = plsc.load_gather(ubuf, [jnp.broadcast_to(col2, (16,))])
            return uv, col2

        def slow_case(_):
            def z3(i, c):
                hist3[pl.ds(i * 16, 16)] = zeros16
                colsum3[pl.ds(i * 16, 16)] = zeros16
                return c
            lax.fori_loop(0, 64, z3, 0)

            def p3(i, c):
                for j in range(_U):
                    ii = i * _U + j
                    u = ubuf[pl.ds(ii * 16, 16)]
                    d1 = lax.shift_right_logical(u, 21)
                    d2 = lax.bitwise_and(
                        lax.shift_right_logical(u, 10), 0x7FF)
                    d3 = lax.bitwise_and(u, 0x3FF)
                    m = (d1 == b1) & (d2 == b2)
                    colv = lane + ii * 16
                    plsc.addupdate_scatter(hist3, [d3], ones, mask=m)
                    plsc.addupdate_scatter(colsum3, [d3], colv, mask=m)
                return c
            lax.fori_loop(0, _NV // _U, p3, 0)
            b3, cb3, cnt3, col3 = scan_hist(hist3, colsum3, 1024, k3, False)
            ustar = lax.bitwise_or(
                lax.bitwise_or(lax.shift_left(b1, 21), lax.shift_left(b2, 10)),
                b3)
            ustar_v = jnp.broadcast_to(ustar, (16,))

            def tie_case(_):
                m0 = k3 - cb3 - 1

                def lbody(i, carry):
                    eqcnt, ans = carry
                    u = ubuf[pl.ds(i * 16, 16)]
                    meq = u == ustar_v
                    csv = plsc.cumsum(meq.astype(jnp.int32))
                    sel = meq & ((csv + eqcnt) == (m0 + 1))
                    colv = lane + i * 16
                    ans = jnp.maximum(ans, jnp.where(sel, colv, -1))
                    eqcnt = eqcnt + plsc.all_reduce_population_count(meq)
                    return eqcnt, ans
                _, ans = lax.fori_loop(
                    0, _NV, lbody,
                    (zeros16, jnp.full((16,), -1, jnp.int32)))
                return jnp.max(ans)

            col = lax.cond(cnt3 == 1, lambda _: col3, tie_case, 0)
            return ustar_v, col

        with jax.named_scope("resolve"):
            uv, col = lax.cond(cnt2 == 1, fast_case, slow_case, 0)
        bits_v = jnp.where(uv < 0, lax.bitwise_xor(uv, cimin),
                           lax.bitwise_not(uv))
        val_v = lax.bitcast_convert_type(bits_v, jnp.float32)
        val_acc = jnp.where(lane == r, val_v, val_acc)
        idx_acc = jnp.where(lane == r, col, idx_acc)
        return val_acc, idx_acc

    val_acc, idx_acc = lax.fori_loop(
        0, _ROWS_PER_W, row_body,
        (jnp.zeros((16,), jnp.float32), jnp.zeros((16,), jnp.int32)))

    resv[...] = val_acc
    resi[...] = idx_acc
    pltpu.sync_copy(resv, val_hbm.at[wid])
    pltpu.sync_copy(resi, idx_hbm.at[wid])


def _kth_select_sc(x, k_arr):
    mesh = plsc.VectorSubcoreMesh(core_axis_name="c", subcore_axis_name="s")
    f = pl.kernel(
        _sc_body,
        out_type=[
            jax.ShapeDtypeStruct((_NW, 16), jnp.float32),
            jax.ShapeDtypeStruct((_NW, 16), jnp.int32),
        ],
        mesh=mesh,
        compiler_params=pltpu.CompilerParams(needs_layout_passes=False),
        scratch_types=[
            pltpu.VMEM((_N,), jnp.float32),
            pltpu.VMEM((_N,), jnp.int32),
            pltpu.VMEM((2048,), jnp.int32),
            pltpu.VMEM((2048,), jnp.int32),
            pltpu.VMEM((2048,), jnp.int32),
            pltpu.VMEM((1024,), jnp.int32),
            pltpu.VMEM((1024,), jnp.int32),
            pltpu.VMEM((16,), jnp.int32),
            pltpu.VMEM((16,), jnp.float32),
            pltpu.VMEM((16,), jnp.int32),
        ],
    )
    return f(x, k_arr)


def _tc_select_body(k_ref, x_ref, val_ref, idx_ref):
    _INT_MIN = jnp.int32(-2147483648)
    xb = x_ref[...]
    b = lax.bitcast_convert_type(xb, jnp.int32)
    asr = lax.shift_right_arithmetic(b, jnp.int32(31))
    u = lax.bitwise_xor(b, lax.bitwise_or(asr, _INT_MIN))
    s = lax.bitwise_xor(u, _INT_MIN)
    k = k_ref[0]

    def val_step(it, p):
        j = 31 - it
        c_u = lax.bitwise_or(p, lax.shift_left(jnp.int32(1), j) - 1)
        c_s = lax.bitwise_xor(c_u, _INT_MIN)
        cnt = jnp.sum((s <= c_s).astype(jnp.int32), axis=1, keepdims=True)
        bit = lax.shift_left(jnp.int32(1), j)
        return jnp.where(cnt >= k, p, lax.bitwise_or(p, bit))

    p = lax.fori_loop(0, 32, val_step, jnp.zeros((_TC_BLOCK, 1), jnp.int32))

    s_star = lax.bitwise_xor(p, _INT_MIN)
    eq = s == s_star
    cnt_less = jnp.sum((s < s_star).astype(jnp.int32), axis=1, keepdims=True)
    m1 = k - cnt_less
    cols = lax.broadcasted_iota(jnp.int32, (_TC_BLOCK, _N), 1)

    def idx_step(it, q):
        j = 14 - it
        c_col = lax.bitwise_or(q, lax.shift_left(jnp.int32(1), j) - 1)
        cnt2 = jnp.sum((eq & (cols <= c_col)).astype(jnp.int32), axis=1,
                       keepdims=True)
        bit = lax.shift_left(jnp.int32(1), j)
        return jnp.where(cnt2 >= m1, q, lax.bitwise_or(q, bit))

    q = lax.fori_loop(0, 15, idx_step, jnp.zeros((_TC_BLOCK, 1), jnp.int32))

    bits = jnp.where(p < 0, lax.bitwise_xor(p, _INT_MIN), lax.bitwise_not(p))
    val_ref[...] = lax.bitcast_convert_type(bits, jnp.float32)
    idx_ref[...] = q


def _kth_select_tc(x, k_arr):
    nb = _TC_ROWS // _TC_BLOCK
    off = _SC_ROWS // _TC_BLOCK
    return pl.pallas_call(
        _tc_select_body,
        grid=(nb,),
        in_specs=[
            pl.BlockSpec(memory_space=pltpu.SMEM),
            pl.BlockSpec((_TC_BLOCK, _N), lambda i: (i + off, 0)),
        ],
        out_specs=[
            pl.BlockSpec((_TC_BLOCK, 1), lambda i: (i, 0)),
            pl.BlockSpec((_TC_BLOCK, 1), lambda i: (i, 0)),
        ],
        out_shape=[
            jax.ShapeDtypeStruct((_TC_ROWS, 1), jnp.float32),
            jax.ShapeDtypeStruct((_TC_ROWS, 1), jnp.int32),
        ],
    )(k_arr, x)

def kernel(x, k, dim, keepdim, values, indices):
    k_arr = jnp.full((16,), jnp.asarray(k, jnp.int32))
    vals, idxs = _kth_select_sc(x, k_arr)
    tc_vals, tc_idxs = _kth_select_tc(x, k_arr[:1])
    kth_val = jnp.concatenate(
        [vals[:, :_ROWS_PER_W].reshape(_SC_ROWS, 1), tc_vals], axis=0)
    kth_idx = jnp.concatenate(
        [idxs[:, :_ROWS_PER_W].reshape(_SC_ROWS, 1), tc_idxs], axis=0)
    zero = (jnp.asarray(dim, jnp.int32) - 1) + (
        jnp.asarray(keepdim).astype(jnp.int32) - 1)
    kth_val = (kth_val + zero.astype(kth_val.dtype)).astype(values.dtype)
    kth_idx = (kth_idx + zero).astype(indices.dtype)
    return kth_val, kth_idx


# speedup vs baseline: 1.5639x; 1.1186x over previous
import jax
import jax.numpy as jnp
from jax import lax
from jax.experimental import pallas as pl
from jax.experimental.pallas import tpu as pltpu
from jax.experimental.pallas import tpu_sc as plsc

_N = 32768
_NV = _N // 16
_NC = 2
_NW = 32
_ROWS_PER_W = 2
_SC_ROWS = _NW * _ROWS_PER_W
_TC_ROWS = 128 - _SC_ROWS
_TC_BLOCK = 8
_U = 4
_US = 4


def _sc_body(x_hbm, k_hbm, val_hbm, idx_hbm,
             xrow, ubuf, hist1, hist2, colsum2, hist3, colsum3,
             kbuf, resv, resi):
    imin = jnp.int32(-2147483648)
    lane = lax.iota(jnp.int32, 16)
    c31 = jnp.full((16,), 31, jnp.int32)
    cimin = jnp.full((16,), imin, jnp.int32)
    ones = jnp.ones((16,), jnp.int32)
    zeros16 = jnp.zeros((16,), jnp.int32)

    wid = lax.axis_index("s") * _NC + lax.axis_index("c")

    pltpu.sync_copy(k_hbm, kbuf)
    kv = kbuf[...]

    def zinit(i, c):
        hist1[pl.ds(i * 16, 16)] = zeros16
        hist2[pl.ds(i * 16, 16)] = zeros16
        colsum2[pl.ds(i * 16, 16)] = zeros16
        return c
    lax.fori_loop(0, 128, zinit, 0)

    def scan_hist(href, csref, nbins, kcur, zero_after):
        def body(i, carry):
            tot_carry, b_acc, cb_acc, cnt_acc, col_acc = carry
            for j in range(_US):
                ii = i * _US + j
                hv = href[pl.ds(ii * 16, 16)]
                if csref is not None:
                    cv = csref[pl.ds(ii * 16, 16)]
                if zero_after:
                    href[pl.ds(ii * 16, 16)] = zeros16
                    if csref is not None:
                        csref[pl.ds(ii * 16, 16)] = zeros16
                cs = plsc.cumsum(hv)
                tot = cs + tot_carry
                excl = tot - hv
                sel = (tot >= kcur) & (excl < kcur)
                gidx = lane + ii * 16
                b_acc = b_acc + jnp.where(sel, gidx, 0)
                cb_acc = cb_acc + jnp.where(sel, excl, 0)
                cnt_acc = cnt_acc + jnp.where(sel, hv, 0)
                if csref is not None:
                    col_acc = col_acc + jnp.where(sel, cv, 0)
                tot_carry = tot_carry + jnp.max(cs)
            return tot_carry, b_acc, cb_acc, cnt_acc, col_acc
        _, b_acc, cb_acc, cnt_acc, col_acc = lax.fori_loop(
            0, nbins // (16 * _US), body, (zeros16,) * 5)
        return (jnp.max(b_acc), jnp.max(cb_acc), jnp.max(cnt_acc),
                jnp.max(col_acc))

    def row_body(r, accs):
        val_acc, idx_acc = accs
        with jax.named_scope("dma_row"):
            pltpu.sync_copy(x_hbm.at[wid * _ROWS_PER_W + r], xrow)

        def p1(i, c):
            for j in range(_U):
                ii = i * _U + j
                xv = xrow[pl.ds(ii * 16, 16)]
                b = lax.bitcast_convert_type(xv, jnp.int32)
                asr = lax.shift_right_arithmetic(b, c31)
                u = lax.bitwise_xor(b, lax.bitwise_or(asr, cimin))
                ubuf[pl.ds(ii * 16, 16)] = u
                plsc.addupdate_scatter(
                    hist1, [lax.shift_right_logical(u, 21)], ones)
            return c
        with jax.named_scope("p1"):
            lax.fori_loop(0, _NV // _U, p1, 0)
        with jax.named_scope("scan1"):
            b1, cb1, cnt1, _ = scan_hist(hist1, None, 2048, kv, True)
        k2 = kv - cb1

        def p2(i, c):
            for j in range(_U):
                ii = i * _U + j
                u = ubuf[pl.ds(ii * 16, 16)]
                d1 = lax.shift_right_logical(u, 21)
                d2 = lax.bitwise_and(lax.shift_right_logical(u, 10), 0x7FF)
                m = d1 == b1
                colv = lane + ii * 16
                plsc.addupdate_scatter(hist2, [d2], ones, mask=m)
                plsc.addupdate_scatter(colsum2, [d2], colv, mask=m)
            return c
        with jax.named_scope("p2"):
            lax.fori_loop(0, _NV // _U, p2, 0)
        with jax.named_scope("scan2"):
            b2, cb2, cnt2, col2 = scan_hist(hist2, colsum2, 2048, k2, True)
        k3 = k2 - cb2

        def fast_case(_):
            uv = plsc.load_gather(ubuf, [jnp.broadcast_to(col2, (16,))])
            return uv, col2

        def slow_case(_):
            def z3(i, c):
                hist3[pl.ds(i * 16, 16)] = zeros16
                colsum3[pl.ds(i * 16, 16)] = zeros16
                return c
            lax.fori_loop(0, 64, z3, 0)

            def p3(i, c):
                for j in range(_U):
                    ii = i * _U + j
                    u = ubuf[pl.ds(ii * 16, 16)]
                    d1 = lax.shift_right_logical(u, 21)
                    d2 = lax.bitwise_and(
                        lax.shift_right_logical(u, 10), 0x7FF)
                    d3 = lax.bitwise_and(u, 0x3FF)
                    m = (d1 == b1) & (d2 == b2)
                    colv = lane + ii * 16
                    plsc.addupdate_scatter(hist3, [d3], ones, mask=m)
                    plsc.addupdate_scatter(colsum3, [d3], colv, mask=m)
                return c
            lax.fori_loop(0, _NV // _U, p3, 0)
            b3, cb3, cnt3, col3 = scan_hist(hist3, colsum3, 1024, k3, False)
            ustar = lax.bitwise_or(
                lax.bitwise_or(lax.shift_left(b1, 21), lax.shift_left(b2, 10)),
                b3)
            ustar_v = jnp.broadcast_to(ustar, (16,))

            def tie_case(_):
                m0 = k3 - cb3 - 1

                def lbody(i, carry):
                    eqcnt, ans = carry
                    u = ubuf[pl.ds(i * 16, 16)]
                    meq = u == ustar_v
                    csv = plsc.cumsum(meq.astype(jnp.int32))
                    sel = meq & ((csv + eqcnt) == (m0 + 1))
                    colv = lane + i * 16
                    ans = jnp.maximum(ans, jnp.where(sel, colv, -1))
                    eqcnt = eqcnt + plsc.all_reduce_population_count(meq)
                    return eqcnt, ans
                _, ans = lax.fori_loop(
                    0, _NV, lbody,
                    (zeros16, jnp.full((16,), -1, jnp.int32)))
                return jnp.max(ans)

            col = lax.cond(cnt3 == 1, lambda _: col3, tie_case, 0)
            return ustar_v, col

        with jax.named_scope("resolve"):
            uv, col = lax.cond(cnt2 == 1, fast_case, slow_case, 0)
        bits_v = jnp.where(uv < 0, lax.bitwise_xor(uv, cimin),
                           lax.bitwise_not(uv))
        val_v = lax.bitcast_convert_type(bits_v, jnp.float32)
        val_acc = jnp.where(lane == r, val_v, val_acc)
        idx_acc = jnp.where(lane == r, col, idx_acc)
        return val_acc, idx_acc

    val_acc, idx_acc = lax.fori_loop(
        0, _ROWS_PER_W, row_body,
        (jnp.zeros((16,), jnp.float32), jnp.zeros((16,), jnp.int32)))

    resv[...] = val_acc
    resi[...] = idx_acc
    pltpu.sync_copy(resv, val_hbm.at[wid])
    pltpu.sync_copy(resi, idx_hbm.at[wid])


def _kth_select_sc(x, k_arr):
    mesh = plsc.VectorSubcoreMesh(core_axis_name="c", subcore_axis_name="s")
    f = pl.kernel(
        _sc_body,
        out_type=[
            jax.ShapeDtypeStruct((_NW, 16), jnp.float32),
            jax.ShapeDtypeStruct((_NW, 16), jnp.int32),
        ],
        mesh=mesh,
        compiler_params=pltpu.CompilerParams(needs_layout_passes=False),
        scratch_types=[
            pltpu.VMEM((_N,), jnp.float32),
            pltpu.VMEM((_N,), jnp.int32),
            pltpu.VMEM((2048,), jnp.int32),
            pltpu.VMEM((2048,), jnp.int32),
            pltpu.VMEM((2048,), jnp.int32),
            pltpu.VMEM((1024,), jnp.int32),
            pltpu.VMEM((1024,), jnp.int32),
            pltpu.VMEM((16,), jnp.int32),
            pltpu.VMEM((16,), jnp.float32),
            pltpu.VMEM((16,), jnp.int32),
        ],
    )
    return f(x, k_arr)


def _tc_select_body(k_ref, x_ref, val_ref, idx_ref):
    _INT_MIN = jnp.int32(-2147483648)
    xb = x_ref[...]
    b = lax.bitcast_convert_type(xb, jnp.int32)
    asr = lax.shift_right_arithmetic(b, jnp.int32(31))
    u = lax.bitwise_xor(b, lax.bitwise_or(asr, _INT_MIN))
    s = lax.bitwise_xor(u, _INT_MIN)
    k = k_ref[0]

    def val_step(it, p):
        j = 31 - it
        c_u = lax.bitwise_or(p, lax.shift_left(jnp.int32(1), j) - 1)
        c_s = lax.bitwise_xor(c_u, _INT_MIN)
        cnt = jnp.sum((s <= c_s).astype(jnp.int32), axis=1, keepdims=True)
        bit = lax.shift_left(jnp.int32(1), j)
        return jnp.where(cnt >= k, p, lax.bitwise_or(p, bit))

    p = lax.fori_loop(0, 32, val_step, jnp.zeros((_TC_BLOCK, 1), jnp.int32))

    s_star = lax.bitwise_xor(p, _INT_MIN)
    eq = s == s_star
    cnt_less = jnp.sum((s < s_star).astype(jnp.int32), axis=1, keepdims=True)
    m1 = k - cnt_less
    cols = lax.broadcasted_iota(jnp.int32, (_TC_BLOCK, _N), 1)

    def cheap_idx(_):
        return jnp.min(jnp.where(eq, cols, jnp.int32(1 << 30)), axis=1,
                       keepdims=True)

    def full_idx(_):
        def idx_step(it, q):
            j = 14 - it
            c_col = lax.bitwise_or(q, lax.shift_left(jnp.int32(1), j) - 1)
            cnt2 = jnp.sum((eq & (cols <= c_col)).astype(jnp.int32), axis=1,
                           keepdims=True)
            bit = lax.shift_left(jnp.int32(1), j)
            return jnp.where(cnt2 >= m1, q, lax.bitwise_or(q, bit))
        return lax.fori_loop(0, 15, idx_step,
                             jnp.zeros((_TC_BLOCK, 1), jnp.int32))

    q = lax.cond(jnp.all(m1 == 1), cheap_idx, full_idx, 0)

    bits = jnp.where(p < 0, lax.bitwise_xor(p, _INT_MIN), lax.bitwise_not(p))
    val_ref[...] = lax.bitcast_convert_type(bits, jnp.float32)
    idx_ref[...] = q


def _kth_select_tc(x, k_arr):
    nb = _TC_ROWS // _TC_BLOCK
    off = _SC_ROWS // _TC_BLOCK
    return pl.pallas_call(
        _tc_select_body,
        grid=(nb,),
        in_specs=[
            pl.BlockSpec(memory_space=pltpu.SMEM),
            pl.BlockSpec((_TC_BLOCK, _N), lambda i: (i + off, 0)),
        ],
        out_specs=[
            pl.BlockSpec((_TC_BLOCK, 1), lambda i: (i, 0)),
            pl.BlockSpec((_TC_BLOCK, 1), lambda i: (i, 0)),
        ],
        out_shape=[
            jax.ShapeDtypeStruct((_TC_ROWS, 1), jnp.float32),
            jax.ShapeDtypeStruct((_TC_ROWS, 1), jnp.int32),
        ],
    )(k_arr, x)

def kernel(x, k, dim, keepdim, values, indices):
    k_arr = jnp.full((16,), jnp.asarray(k, jnp.int32))
    vals, idxs = _kth_select_sc(x, k_arr)
    tc_vals, tc_idxs = _kth_select_tc(x, k_arr[:1])
    kth_val = jnp.concatenate(
        [vals[:, :_ROWS_PER_W].reshape(_SC_ROWS, 1), tc_vals], axis=0)
    kth_idx = jnp.concatenate(
        [idxs[:, :_ROWS_PER_W].reshape(_SC_ROWS, 1), tc_idxs], axis=0)
    zero = (jnp.asarray(dim, jnp.int32) - 1) + (
        jnp.asarray(keepdim).astype(jnp.int32) - 1)
    kth_val = (kth_val + zero.astype(kth_val.dtype)).astype(values.dtype)
    kth_idx = (kth_idx + zero).astype(indices.dtype)
    return kth_val, kth_idx
